# baseline (speedup 1.0000x reference)
import functools

import jax
import jax.numpy as jnp
from jax import lax
from jax.experimental import pallas as pl
from jax.experimental.pallas import tpu as pltpu
from jax.experimental.pallas import tpu_sc as plsc

VOCAB = 1000000
D = 64
B = 4096
L = 200

NC = 2
NS = 16
NW = NC * NS
BW = B // NW
NPAIR = L // 2

TCHUNK = 1024


def _repack_body(wt_ref, o_ref):
    r = lax.broadcasted_iota(jnp.int32, (D, D), 0)
    c = lax.broadcasted_iota(jnp.int32, (D, D), 1)
    eye = (r == c).astype(jnp.float32)
    t = lax.dot_general(
        wt_ref[...], eye, (((0,), (0,)), ((), ())),
        preferred_element_type=jnp.float32,
    )
    o_ref[...] = jnp.concatenate([t, t], axis=1)


def _emb_body(xt_hbm, w_hbm, out_hbm, idx_v, stg0, stg1, str0, str1,
              gsem0, gsem1, wsem0, wsem1):
    wid = lax.axis_index("s") * NC + lax.axis_index("c")
    b0 = wid * BW
    pltpu.sync_copy(xt_hbm.at[pl.ds(0, L), pl.ds(b0, BW)], idx_v)
    iota = lax.iota(jnp.int32, 16)

    def fire_gather(l, stg, gsem):
        pltpu.async_copy(w_hbm.at[idx_v.at[l]], stg, gsem)

    def drain_gather(stg, gsem):
        pltpu.make_async_copy(w_hbm.at[idx_v.at[0]], stg, gsem).wait()

    def transpose_l(stg, strip):
        def bg_body(bg, carry):
            b16 = bg * 16 + iota

            @plsc.parallel_loop(0, D, 1, unroll=16)
            def c_body(c):
                cp16 = (c + b16) & 63
                val = plsc.load_gather(stg, [b16, cp16])
                plsc.store_scatter(strip, [cp16, b16], val)
            return carry

        lax.fori_loop(0, BW // 16, bg_body, 0)

    def step(q, carry):
        l0 = 2 * q
        drain_gather(stg0, gsem0)
        fire_gather(l0 + 1, stg1, gsem1)

        @pl.when(q > 0)
        def _():
            pltpu.make_async_copy(
                str0, out_hbm.at[l0, :, pl.ds(b0, BW)], wsem0
            ).wait()
        transpose_l(stg0, str0)
        pltpu.async_copy(str0, out_hbm.at[l0, :, pl.ds(b0, BW)], wsem0)

        @pl.when(q + 1 < NPAIR)
        def _():
            fire_gather(l0 + 2, stg0, gsem0)

        drain_gather(stg1, gsem1)

        @pl.when(q > 0)
        def _():
            pltpu.make_async_copy(
                str1, out_hbm.at[l0, :, pl.ds(b0, BW)], wsem1
            ).wait()
        transpose_l(stg1, str1)
        pltpu.async_copy(str1, out_hbm.at[l0 + 1, :, pl.ds(b0, BW)], wsem1)
        return carry

    fire_gather(0, stg0, gsem0)
    lax.fori_loop(0, NPAIR, step, 0)
    pltpu.make_async_copy(str0, out_hbm.at[0, :, pl.ds(b0, BW)], wsem0).wait()
    pltpu.make_async_copy(str1, out_hbm.at[0, :, pl.ds(b0, BW)], wsem1).wait()


@jax.jit
def _embedding_lookup(xt, Wt):
    n_chunks = (VOCAB + TCHUNK - 1) // TCHUNK
    Wp = pl.pallas_call(
        _repack_body,
        grid=(n_chunks,),
        in_specs=[pl.BlockSpec((D, TCHUNK), lambda i: (0, i))],
        out_specs=pl.BlockSpec((TCHUNK, 2 * D), lambda i: (i, 0)),
        out_shape=jax.ShapeDtypeStruct((VOCAB, 2 * D), jnp.float32),
    )(Wt)

    f = functools.partial(
        pl.kernel,
        mesh=plsc.VectorSubcoreMesh(core_axis_name="c", subcore_axis_name="s"),
        out_type=jax.ShapeDtypeStruct((L, D, B), jnp.float32),
        scratch_types=[
            pltpu.VMEM((L, BW), jnp.int32),
            pltpu.VMEM((BW, 128), jnp.float32),
            pltpu.VMEM((BW, 128), jnp.float32),
            pltpu.VMEM((D, BW), jnp.float32),
            pltpu.VMEM((D, BW), jnp.float32),
            pltpu.SemaphoreType.DMA,
            pltpu.SemaphoreType.DMA,
            pltpu.SemaphoreType.DMA,
            pltpu.SemaphoreType.DMA,
        ],
        compiler_params=pltpu.CompilerParams(
            use_tc_tiling_on_sc=True, needs_layout_passes=False
        ),
    )(_emb_body)
    return f(xt, Wp)


def kernel(x, W):
    out_t = _embedding_lookup(x.T, W.T)
    return jnp.transpose(out_t, (2, 0, 1))


# speedup vs baseline: 1.4196x; 1.4196x over previous
import functools

import jax
import jax.numpy as jnp
from jax import lax
from jax.experimental import pallas as pl
from jax.experimental.pallas import tpu as pltpu
from jax.experimental.pallas import tpu_sc as plsc

VOCAB = 1000000
D = 64
B = 4096
L = 200

NC = 2
NS = 16
NW = NC * NS
BW = B // NW
NPAIR = L // 2

TCHUNK = 4096


def _repack_body(wt_ref, o_ref):
    r = lax.broadcasted_iota(jnp.int32, (D, D), 0)
    c = lax.broadcasted_iota(jnp.int32, (D, D), 1)
    eye = (r == c).astype(jnp.float32)
    t = lax.dot_general(
        wt_ref[...], eye, (((0,), (0,)), ((), ())),
        precision=lax.Precision.HIGHEST,
        preferred_element_type=jnp.float32,
    )
    o_ref[...] = jnp.concatenate([t, t], axis=1)


def _emb_body(xt_hbm, w_hbm, out_hbm, idx_v, stg0, stg1, str0, str1,
              gsem0, gsem1, wsem0, wsem1):
    wid = lax.axis_index("s") * NC + lax.axis_index("c")
    b0 = wid * BW
    pltpu.sync_copy(xt_hbm.at[pl.ds(0, L), pl.ds(b0, BW)], idx_v)
    iota = lax.iota(jnp.int32, 16)

    def fire_gather(l, stg, gsem):
        pltpu.async_copy(w_hbm.at[idx_v.at[l]], stg, gsem)

    def drain_gather(stg, gsem):
        pltpu.make_async_copy(w_hbm.at[idx_v.at[0]], stg, gsem).wait()

    def transpose_l(stg, strip):
        def bg_body(bg, carry):
            b16 = bg * 16 + iota

            @plsc.parallel_loop(0, D, 1, unroll=16)
            def c_body(c):
                cp16 = (c + b16) & 63
                val = plsc.load_gather(stg, [b16, cp16])
                plsc.store_scatter(strip, [cp16, b16], val)
            return carry

        lax.fori_loop(0, BW // 16, bg_body, 0)

    def step(q, carry):
        l0 = 2 * q
        drain_gather(stg0, gsem0)
        fire_gather(l0 + 1, stg1, gsem1)

        @pl.when(q > 0)
        def _():
            pltpu.make_async_copy(
                str0, out_hbm.at[l0, :, pl.ds(b0, BW)], wsem0
            ).wait()
        transpose_l(stg0, str0)
        pltpu.async_copy(str0, out_hbm.at[l0, :, pl.ds(b0, BW)], wsem0)

        @pl.when(q + 1 < NPAIR)
        def _():
            fire_gather(l0 + 2, stg0, gsem0)

        drain_gather(stg1, gsem1)

        @pl.when(q > 0)
        def _():
            pltpu.make_async_copy(
                str1, out_hbm.at[l0, :, pl.ds(b0, BW)], wsem1
            ).wait()
        transpose_l(stg1, str1)
        pltpu.async_copy(str1, out_hbm.at[l0 + 1, :, pl.ds(b0, BW)], wsem1)
        return carry

    fire_gather(0, stg0, gsem0)
    lax.fori_loop(0, NPAIR, step, 0)
    pltpu.make_async_copy(str0, out_hbm.at[0, :, pl.ds(b0, BW)], wsem0).wait()
    pltpu.make_async_copy(str1, out_hbm.at[0, :, pl.ds(b0, BW)], wsem1).wait()


@jax.jit
def _embedding_lookup(xt, Wt):
    n_chunks = (VOCAB + TCHUNK - 1) // TCHUNK
    Wp = pl.pallas_call(
        _repack_body,
        grid=(n_chunks,),
        in_specs=[pl.BlockSpec((D, TCHUNK), lambda i: (0, i))],
        out_specs=pl.BlockSpec((TCHUNK, 2 * D), lambda i: (i, 0)),
        out_shape=jax.ShapeDtypeStruct((VOCAB, 2 * D), jnp.float32),
    )(Wt)

    f = functools.partial(
        pl.kernel,
        mesh=plsc.VectorSubcoreMesh(core_axis_name="c", subcore_axis_name="s"),
        out_type=jax.ShapeDtypeStruct((L, D, B), jnp.float32),
        scratch_types=[
            pltpu.VMEM((L, BW), jnp.int32),
            pltpu.VMEM((BW, 128), jnp.float32),
            pltpu.VMEM((BW, 128), jnp.float32),
            pltpu.VMEM((D, BW), jnp.float32),
            pltpu.VMEM((D, BW), jnp.float32),
            pltpu.SemaphoreType.DMA,
            pltpu.SemaphoreType.DMA,
            pltpu.SemaphoreType.DMA,
            pltpu.SemaphoreType.DMA,
        ],
        compiler_params=pltpu.CompilerParams(
            use_tc_tiling_on_sc=True, needs_layout_passes=False
        ),
    )(_emb_body)
    return f(xt, Wp)


def kernel(x, W):
    out_t = _embedding_lookup(x.T, W.T)
    return jnp.transpose(out_t, (2, 0, 1))


# speedup vs baseline: 1.5094x; 1.0633x over previous
import functools

import jax
import jax.numpy as jnp
from jax import lax
from jax.experimental import pallas as pl
from jax.experimental.pallas import tpu as pltpu
from jax.experimental.pallas import tpu_sc as plsc

VOCAB = 1000000
D = 64
B = 4096
L = 200

NC = 2
NS = 16
NW = NC * NS
BW = B // NW
NPAIR = L // 2

TCHUNK = 8192


def _repack_body(wt_ref, o_ref):
    r = lax.broadcasted_iota(jnp.int32, (D, D), 0)
    c = lax.broadcasted_iota(jnp.int32, (D, D), 1)
    eye = (r == c).astype(jnp.float32)
    t = lax.dot_general(
        wt_ref[...], eye, (((0,), (0,)), ((), ())),
        precision=lax.Precision.HIGHEST,
        preferred_element_type=jnp.float32,
    )
    o_ref[...] = jnp.concatenate([t, t], axis=1)


def _emb_body(xt_hbm, w_hbm, out_hbm, idx_v, stg0, stg1, str0, str1,
              gsem0, gsem1, wsem0, wsem1):
    wid = lax.axis_index("s") * NC + lax.axis_index("c")
    b0 = wid * BW
    pltpu.sync_copy(xt_hbm.at[pl.ds(0, L), pl.ds(b0, BW)], idx_v)
    iota = lax.iota(jnp.int32, 16)

    def fire_gather(l, stg, gsem):
        pltpu.async_copy(w_hbm.at[idx_v.at[l]], stg, gsem)

    def drain_gather(stg, gsem):
        pltpu.make_async_copy(w_hbm.at[idx_v.at[0]], stg, gsem).wait()

    def transpose_l(stg, strip):
        def bg_body(bg, carry):
            b16 = bg * 16 + iota

            @plsc.parallel_loop(0, D, 1, unroll=16)
            def c_body(c):
                cp16 = (c + b16) & 63
                val = plsc.load_gather(stg, [b16, cp16])
                plsc.store_scatter(strip, [cp16, b16], val)
            return carry

        lax.fori_loop(0, BW // 16, bg_body, 0)

    def step(q, carry):
        l0 = 2 * q
        drain_gather(stg0, gsem0)
        fire_gather(l0 + 1, stg1, gsem1)

        @pl.when(q > 0)
        def _():
            pltpu.make_async_copy(
                str0, out_hbm.at[l0, :, pl.ds(b0, BW)], wsem0
            ).wait()
        transpose_l(stg0, str0)
        pltpu.async_copy(str0, out_hbm.at[l0, :, pl.ds(b0, BW)], wsem0)

        @pl.when(q + 1 < NPAIR)
        def _():
            fire_gather(l0 + 2, stg0, gsem0)

        drain_gather(stg1, gsem1)

        @pl.when(q > 0)
        def _():
            pltpu.make_async_copy(
                str1, out_hbm.at[l0, :, pl.ds(b0, BW)], wsem1
            ).wait()
        transpose_l(stg1, str1)
        pltpu.async_copy(str1, out_hbm.at[l0 + 1, :, pl.ds(b0, BW)], wsem1)
        return carry

    fire_gather(0, stg0, gsem0)
    lax.fori_loop(0, NPAIR, step, 0)
    pltpu.make_async_copy(str0, out_hbm.at[0, :, pl.ds(b0, BW)], wsem0).wait()
    pltpu.make_async_copy(str1, out_hbm.at[0, :, pl.ds(b0, BW)], wsem1).wait()


@jax.jit
def _embedding_lookup(xt, Wt):
    n_chunks = (VOCAB + TCHUNK - 1) // TCHUNK
    Wp = pl.pallas_call(
        _repack_body,
        grid=(n_chunks,),
        in_specs=[pl.BlockSpec((D, TCHUNK), lambda i: (0, i))],
        out_specs=pl.BlockSpec((TCHUNK, 2 * D), lambda i: (i, 0)),
        out_shape=jax.ShapeDtypeStruct((VOCAB, 2 * D), jnp.float32),
    )(Wt)

    f = functools.partial(
        pl.kernel,
        mesh=plsc.VectorSubcoreMesh(core_axis_name="c", subcore_axis_name="s"),
        out_type=jax.ShapeDtypeStruct((L, D, B), jnp.float32),
        scratch_types=[
            pltpu.VMEM((L, BW), jnp.int32),
            pltpu.VMEM((BW, 128), jnp.float32),
            pltpu.VMEM((BW, 128), jnp.float32),
            pltpu.VMEM((D, BW), jnp.float32),
            pltpu.VMEM((D, BW), jnp.float32),
            pltpu.SemaphoreType.DMA,
            pltpu.SemaphoreType.DMA,
            pltpu.SemaphoreType.DMA,
            pltpu.SemaphoreType.DMA,
        ],
        compiler_params=pltpu.CompilerParams(
            use_tc_tiling_on_sc=True, needs_layout_passes=False
        ),
    )(_emb_body)
    return f(xt, Wp)


def kernel(x, W):
    out_t = _embedding_lookup(x.T, W.T)
    return jnp.transpose(out_t, (2, 0, 1))
